# baseline (speedup 1.0000x reference)
import jax
import jax.numpy as jnp
from jax.experimental import pallas as pl
from jax.experimental.pallas import tpu as pltpu

CHUNK = 256
GROUP = 4


def _gelu_exact(t):
    u = t * (1.5957691216057308 + 0.07135481283785332 * t * t)
    return t / (1.0 + jnp.exp(-u))


_PIO2_HI = 1.57079637e0
_PIO2_LO = -4.37113883e-8


def _sincos(x):
    k_i = jnp.round(x * (2.0 / jnp.pi)).astype(jnp.int32)
    kf = k_i.astype(jnp.float32)
    r = (x - kf * _PIO2_HI) - kf * _PIO2_LO
    r2 = r * r
    sin_r = ((-1.9515295891e-4 * r2 + 8.3321608736e-3) * r2
             - 1.6666654611e-1) * r2 * r + r
    cos_r = ((-1.388731625493765e-3 * r2 + 4.16666456e-2) * r2
             - 0.5) * r2 + 1.0
    swap = (k_i & 1) != 0
    sin_base = jnp.where(swap, cos_r, sin_r)
    cos_base = jnp.where(swap, sin_r, cos_r)
    sin_neg = (k_i & 2) != 0
    cos_neg = ((k_i + 1) & 2) != 0
    sin_x = jnp.where(sin_neg, -sin_base, sin_base)
    cos_x = jnp.where(cos_neg, -cos_base, cos_base)
    return sin_x, cos_x


def _bf16(a):
    return a.astype(jnp.bfloat16)


def _dot(a16, b16):
    return jnp.dot(a16, b16, preferred_element_type=jnp.float32)


def _psi_kernel(tri_ref, x_ref, W_omega, b_omega, W_p1, b_p1, W_p2, b_p2,
                scale, W_mag, b_mag, W_q, b_q, W_o1g, colsum, hbias,
                W_o2, b_o2, o_ref, c_om, c_r, c_i, c_m):
    c = pl.program_id(1)
    D = x_ref.shape[-1]

    @pl.when(c == 0)
    def _():
        c_om[...] = jnp.zeros_like(c_om)
        c_r[...] = jnp.zeros_like(c_r)
        c_i[...] = jnp.zeros_like(c_i)
        c_m[...] = jnp.zeros_like(c_m)

    tri = tri_ref[...]
    C = tri_ref.shape[0]
    G = x_ref.shape[1] // C
    cars = (c_om[...], c_r[...], c_i[...], c_m[...])
    for g in range(G):
        xb = x_ref[0, g * C:(g + 1) * C]
        out_g, cars = _sub_chunk(
            xb, tri, cars, W_omega, b_omega, W_p1, b_p1, W_p2, b_p2, scale,
            W_mag, b_mag, W_q, b_q, W_o1g, colsum, hbias, W_o2, b_o2)
        o_ref[0, g * C:(g + 1) * C] = out_g
    c_om[...], c_r[...], c_i[...], c_m[...] = cars


def _sub_chunk(xb, tri, cars, W_omega, b_omega, W_p1, b_p1, W_p2, b_p2,
               scale, W_mag, b_mag, W_q, b_q, W_o1g, colsum, hbias,
               W_o2, b_o2):
    v_om, v_r, v_i, v_m = cars
    D = xb.shape[-1]
    xb16 = _bf16(xb)

    omega = _dot(xb16, W_omega[...]) + b_omega[...]
    mag = jax.nn.sigmoid(_dot(xb16, W_mag[...]) + b_mag[...]) * 5.0
    h1 = _gelu_exact(_dot(xb16, W_p1[...]) + b_p1[...])
    qz = _dot(xb16, W_q[...]) + b_q[...]
    phi_init = _dot(_bf16(h1), W_p2[...]) + b_p2[...]

    a_scale = jnp.abs(scale[...])
    phi = phi_init + (_dot(tri, _bf16(omega)) + v_om) * a_scale
    v_om = v_om + jnp.sum(omega, axis=0, keepdims=True)

    sin_phi, cos_phi = _sincos(phi)

    wc = mag * xb
    tr = wc * cos_phi
    ti = wc * sin_phi
    mem_r = _dot(tri, _bf16(tr)) + v_r
    v_r = v_r + jnp.sum(tr, axis=0, keepdims=True)
    mem_i = _dot(tri, _bf16(ti)) + v_i
    v_i = v_i + jnp.sum(ti, axis=0, keepdims=True)
    cmag = _dot(tri, _bf16(mag)) + v_m
    v_m = v_m + jnp.sum(mag, axis=0, keepdims=True)

    inv_sq = jax.lax.rsqrt(cmag + 1e-8)
    mr = mem_r * inv_sq
    mi = mem_i * inv_sq

    phi_q = phi + qz
    sq, cq = _sincos(phi_q)
    ret_r = mr * cq + mi * sq
    ret_i = mi * cq - mr * sq

    p0 = xb * cos_phi
    p1 = xb * sin_phi
    ssum = (jnp.sum(p0, axis=-1, keepdims=True)
            + jnp.sum(p1, axis=-1, keepdims=True)
            + jnp.sum(ret_r, axis=-1, keepdims=True)
            + jnp.sum(ret_i, axis=-1, keepdims=True))
    s2sum = (jnp.sum(p0 * p0, axis=-1, keepdims=True)
             + jnp.sum(p1 * p1, axis=-1, keepdims=True)
             + jnp.sum(ret_r * ret_r, axis=-1, keepdims=True)
             + jnp.sum(ret_i * ret_i, axis=-1, keepdims=True))
    inv_4d = 1.0 / (4 * D)
    mu = ssum * inv_4d
    var = s2sum * inv_4d - mu * mu
    invstd = jax.lax.rsqrt(var + 1e-5)

    A = (_dot(_bf16(p0), W_o1g[:D])
         + _dot(_bf16(p1), W_o1g[D:2 * D])
         + _dot(_bf16(ret_r), W_o1g[2 * D:3 * D])
         + _dot(_bf16(ret_i), W_o1g[3 * D:]))
    pre = A * invstd - (mu * invstd) * colsum[...] + hbias[...]
    h = _gelu_exact(pre)
    out = xb + _dot(_bf16(h), W_o2[...]) + b_o2[...]
    return out, (v_om, v_r, v_i, v_m)


@jax.jit
def kernel(x, W_omega, b_omega, W_p1, b_p1, W_p2, b_p2, scale, W_mag, b_mag,
           W_q, b_q, ln_g, ln_b, W_o1, b_o1, W_o2, b_o2):
    B, S, D = x.shape
    C = CHUNK if S % CHUNK == 0 else S
    G = GROUP if (S // C) % GROUP == 0 else 1
    NC = S // (C * G)

    tri = jnp.tril(jnp.ones((C, C), jnp.bfloat16))
    bf = lambda a: a.astype(jnp.bfloat16)

    W_o1g = W_o1 * ln_g[:, None]
    colsum = jnp.sum(W_o1g, axis=0).reshape(1, -1)
    hbias = (ln_b @ W_o1 + b_o1).reshape(1, -1)

    row = lambda v: v.reshape(1, -1)
    const = lambda shape: pl.BlockSpec(shape, lambda b, c: (0,) * len(shape))

    grid = (B, NC)
    out = pl.pallas_call(
        _psi_kernel,
        out_shape=jax.ShapeDtypeStruct((B, S, D), jnp.float32),
        grid=grid,
        in_specs=[
            const((C, C)),
            pl.BlockSpec((1, C * G, D), lambda b, c: (b, c, 0)),
            const((D, D)), const((1, D)),
            const((D, D)), const((1, D)),
            const((D, D)), const((1, D)),
            const((1, D)),
            const((D, D)), const((1, D)),
            const((D, D)), const((1, D)),
            const((4 * D, 2 * D)),
            const((1, 2 * D)), const((1, 2 * D)),
            const((2 * D, D)), const((1, D)),
        ],
        out_specs=pl.BlockSpec((1, C * G, D), lambda b, c: (b, c, 0)),
        scratch_shapes=[pltpu.VMEM((1, D), jnp.float32)] * 4,
        compiler_params=pltpu.CompilerParams(
            dimension_semantics=("parallel", "arbitrary"),
            vmem_limit_bytes=100 * 1024 * 1024,
        ),
        name="psi_no_gate",
    )(tri, x, bf(W_omega), row(b_omega), bf(W_p1), row(b_p1), bf(W_p2),
      row(b_p2), row(scale), bf(W_mag), row(b_mag), bf(W_q), row(b_q),
      bf(W_o1g), colsum, hbias, bf(W_o2), row(b_o2))
    return out


# speedup vs baseline: 1.0576x; 1.0576x over previous
import jax
import jax.numpy as jnp
from jax.experimental import pallas as pl
from jax.experimental.pallas import tpu as pltpu

CHUNK = 512
GROUP = 4


def _gelu_exact(t):
    u = t * (1.5957691216057308 + 0.07135481283785332 * t * t)
    return t / (1.0 + jnp.exp(-u))


_PIO2_HI = 1.57079637e0
_PIO2_LO = -4.37113883e-8


def _sincos(x):
    k_i = jnp.round(x * (2.0 / jnp.pi)).astype(jnp.int32)
    kf = k_i.astype(jnp.float32)
    r = (x - kf * _PIO2_HI) - kf * _PIO2_LO
    r2 = r * r
    sin_r = ((-1.9515295891e-4 * r2 + 8.3321608736e-3) * r2
             - 1.6666654611e-1) * r2 * r + r
    cos_r = ((-1.388731625493765e-3 * r2 + 4.16666456e-2) * r2
             - 0.5) * r2 + 1.0
    swap = (k_i & 1) != 0
    sin_base = jnp.where(swap, cos_r, sin_r)
    cos_base = jnp.where(swap, sin_r, cos_r)
    sin_neg = (k_i & 2) != 0
    cos_neg = ((k_i + 1) & 2) != 0
    sin_x = jnp.where(sin_neg, -sin_base, sin_base)
    cos_x = jnp.where(cos_neg, -cos_base, cos_base)
    return sin_x, cos_x


def _bf16(a):
    return a.astype(jnp.bfloat16)


def _dot(a16, b16):
    return jnp.dot(a16, b16, preferred_element_type=jnp.float32)


def _psi_kernel(tri_ref, x_ref, W_omega, b_omega, W_p1, b_p1, W_p2, b_p2,
                scale, W_mag, b_mag, W_q, b_q, W_o1g, colsum, hbias,
                W_o2, b_o2, o_ref, c_om, c_r, c_i, c_m):
    c = pl.program_id(1)
    D = x_ref.shape[-1]

    @pl.when(c == 0)
    def _():
        c_om[...] = jnp.zeros_like(c_om)
        c_r[...] = jnp.zeros_like(c_r)
        c_i[...] = jnp.zeros_like(c_i)
        c_m[...] = jnp.zeros_like(c_m)

    tri = tri_ref[...]
    C = tri_ref.shape[0]
    G = x_ref.shape[1] // C
    cars = (c_om[...], c_r[...], c_i[...], c_m[...])
    for g in range(G):
        xb = x_ref[0, g * C:(g + 1) * C]
        out_g, cars = _sub_chunk(
            xb, tri, cars, W_omega, b_omega, W_p1, b_p1, W_p2, b_p2, scale,
            W_mag, b_mag, W_q, b_q, W_o1g, colsum, hbias, W_o2, b_o2)
        o_ref[0, g * C:(g + 1) * C] = out_g
    c_om[...], c_r[...], c_i[...], c_m[...] = cars


def _sub_chunk(xb, tri, cars, W_omega, b_omega, W_p1, b_p1, W_p2, b_p2,
               scale, W_mag, b_mag, W_q, b_q, W_o1g, colsum, hbias,
               W_o2, b_o2):
    v_om, v_r, v_i, v_m = cars
    D = xb.shape[-1]
    xb16 = _bf16(xb)

    omega = _dot(xb16, W_omega[...]) + b_omega[...]
    mag = jax.nn.sigmoid(_dot(xb16, W_mag[...]) + b_mag[...]) * 5.0
    h1 = _gelu_exact(_dot(xb16, W_p1[...]) + b_p1[...])
    qz = _dot(xb16, W_q[...]) + b_q[...]
    phi_init = _dot(_bf16(h1), W_p2[...]) + b_p2[...]

    a_scale = jnp.abs(scale[...])
    phi = phi_init + (_dot(tri, _bf16(omega)) + v_om) * a_scale
    v_om = v_om + jnp.sum(omega, axis=0, keepdims=True)

    sin_phi, cos_phi = _sincos(phi)

    wc = mag * xb
    tr = wc * cos_phi
    ti = wc * sin_phi
    mem_r = _dot(tri, _bf16(tr)) + v_r
    v_r = v_r + jnp.sum(tr, axis=0, keepdims=True)
    mem_i = _dot(tri, _bf16(ti)) + v_i
    v_i = v_i + jnp.sum(ti, axis=0, keepdims=True)
    cmag = _dot(tri, _bf16(mag)) + v_m
    v_m = v_m + jnp.sum(mag, axis=0, keepdims=True)

    inv_sq = jax.lax.rsqrt(cmag + 1e-8)
    mr = mem_r * inv_sq
    mi = mem_i * inv_sq

    phi_q = phi + qz
    sq, cq = _sincos(phi_q)
    ret_r = mr * cq + mi * sq
    ret_i = mi * cq - mr * sq

    p0 = xb * cos_phi
    p1 = xb * sin_phi
    ssum = (jnp.sum(p0, axis=-1, keepdims=True)
            + jnp.sum(p1, axis=-1, keepdims=True)
            + jnp.sum(ret_r, axis=-1, keepdims=True)
            + jnp.sum(ret_i, axis=-1, keepdims=True))
    s2sum = (jnp.sum(p0 * p0, axis=-1, keepdims=True)
             + jnp.sum(p1 * p1, axis=-1, keepdims=True)
             + jnp.sum(ret_r * ret_r, axis=-1, keepdims=True)
             + jnp.sum(ret_i * ret_i, axis=-1, keepdims=True))
    inv_4d = 1.0 / (4 * D)
    mu = ssum * inv_4d
    var = s2sum * inv_4d - mu * mu
    invstd = jax.lax.rsqrt(var + 1e-5)

    A = (_dot(_bf16(p0), W_o1g[:D])
         + _dot(_bf16(p1), W_o1g[D:2 * D])
         + _dot(_bf16(ret_r), W_o1g[2 * D:3 * D])
         + _dot(_bf16(ret_i), W_o1g[3 * D:]))
    pre = A * invstd - (mu * invstd) * colsum[...] + hbias[...]
    h = _gelu_exact(pre)
    out = xb + _dot(_bf16(h), W_o2[...]) + b_o2[...]
    return out, (v_om, v_r, v_i, v_m)


@jax.jit
def kernel(x, W_omega, b_omega, W_p1, b_p1, W_p2, b_p2, scale, W_mag, b_mag,
           W_q, b_q, ln_g, ln_b, W_o1, b_o1, W_o2, b_o2):
    B, S, D = x.shape
    C = CHUNK if S % CHUNK == 0 else S
    G = GROUP if (S // C) % GROUP == 0 else 1
    NC = S // (C * G)

    tri = jnp.tril(jnp.ones((C, C), jnp.bfloat16))
    bf = lambda a: a.astype(jnp.bfloat16)

    W_o1g = W_o1 * ln_g[:, None]
    colsum = jnp.sum(W_o1g, axis=0).reshape(1, -1)
    hbias = (ln_b @ W_o1 + b_o1).reshape(1, -1)

    row = lambda v: v.reshape(1, -1)
    const = lambda shape: pl.BlockSpec(shape, lambda b, c: (0,) * len(shape))

    grid = (B, NC)
    out = pl.pallas_call(
        _psi_kernel,
        out_shape=jax.ShapeDtypeStruct((B, S, D), jnp.float32),
        grid=grid,
        in_specs=[
            const((C, C)),
            pl.BlockSpec((1, C * G, D), lambda b, c: (b, c, 0)),
            const((D, D)), const((1, D)),
            const((D, D)), const((1, D)),
            const((D, D)), const((1, D)),
            const((1, D)),
            const((D, D)), const((1, D)),
            const((D, D)), const((1, D)),
            const((4 * D, 2 * D)),
            const((1, 2 * D)), const((1, 2 * D)),
            const((2 * D, D)), const((1, D)),
        ],
        out_specs=pl.BlockSpec((1, C * G, D), lambda b, c: (b, c, 0)),
        scratch_shapes=[pltpu.VMEM((1, D), jnp.float32)] * 4,
        compiler_params=pltpu.CompilerParams(
            dimension_semantics=("parallel", "arbitrary"),
            vmem_limit_bytes=100 * 1024 * 1024,
        ),
        name="psi_no_gate",
    )(tri, x, bf(W_omega), row(b_omega), bf(W_p1), row(b_p1), bf(W_p2),
      row(b_p2), row(scale), bf(W_mag), row(b_mag), bf(W_q), row(b_q),
      bf(W_o1g), colsum, hbias, bf(W_o2), row(b_o2))
    return out
